# Initial kernel scaffold; baseline (speedup 1.0000x reference)
import jax
import jax.numpy as jnp
from jax.experimental import pallas as pl


def kernel(Z, R, idx_i, idx_j, params):
    raise NotImplementedError("write your pallas kernel here")


# speedup vs baseline: 5.8839x; 5.8839x over previous
import functools
import math

import jax
import jax.numpy as jnp
from jax import lax
from jax.experimental import pallas as pl
from jax.experimental.pallas import tpu as pltpu
from jax.experimental.pallas import tpu_sc as plsc

_LOG2 = math.log(2.0)
_CUTOFF = 10.0

_F = 128
_K = 64
_NW = 32
_C = 80

_N_TILE = 1000
_E_TILE = 4096


def _ssp(x):
    return jnp.maximum(x, 0.0) + jnp.log1p(jnp.exp(-jnp.abs(x))) - _LOG2


def _residual(x, w1, b1, w2, b2):
    y = _ssp(x)
    y = _ssp(jnp.dot(y, w1, preferred_element_type=jnp.float32) + b1)
    y = jnp.dot(y, w2, preferred_element_type=jnp.float32) + b2
    return x + y


def _sc_setup_body(emb_hbm, zp_hbm, rf_hbm, ii_hbm, jj_hbm,
                   x0_hbm, d2_hbm,
                   zidx_v, iidx_v, jidx_v, xrows_v, rf_v, d2_v, sem):
    c = lax.axis_index("c")
    s = lax.axis_index("s")
    wid = c * 16 + s

    n_at = zp_hbm.shape[0] // _NW
    zbase = wid * n_at

    def zbody(k, carry):
        off = zbase + k * _C
        pltpu.sync_copy(zp_hbm.at[pl.ds(off, _C)], zidx_v)
        pltpu.async_copy(emb_hbm.at[zidx_v], xrows_v, sem).wait()
        pltpu.sync_copy(xrows_v, x0_hbm.at[pl.ds(off, _C)])
        return carry

    lax.fori_loop(0, n_at // _C, zbody, 0)

    pltpu.sync_copy(rf_hbm, rf_v)

    n_e = ii_hbm.shape[0] // _NW
    ebase = wid * n_e
    pltpu.sync_copy(ii_hbm.at[pl.ds(ebase, n_e)], iidx_v)
    pltpu.sync_copy(jj_hbm.at[pl.ds(ebase, n_e)], jidx_v)

    def ebody(g, carry):
        sl = pl.ds(g * 16, 16)
        ii4 = iidx_v[sl] * 4
        jj4 = jidx_v[sl] * 4
        acc = jnp.zeros((16,), jnp.float32)
        for cc in range(3):
            a = plsc.load_gather(rf_v, [ii4 + cc])
            b = plsc.load_gather(rf_v, [jj4 + cc])
            dxyz = b - a
            acc = acc + dxyz * dxyz
        d2_v[sl] = acc
        return carry

    lax.fori_loop(0, n_e // 16, ebody, 0)
    pltpu.sync_copy(d2_v, d2_hbm.at[pl.ds(ebase, n_e)])


def _sc_setup(emb, zp, rf, idx_i, idx_j):
    n_pad = zp.shape[0]
    e = idx_i.shape[0]
    mesh = plsc.VectorSubcoreMesh(core_axis_name="c", subcore_axis_name="s")
    fn = pl.kernel(
        _sc_setup_body,
        out_type=[
            jax.ShapeDtypeStruct((n_pad, _F), jnp.float32),
            jax.ShapeDtypeStruct((e,), jnp.float32),
        ],
        mesh=mesh,
        scratch_types=[
            pltpu.VMEM((_C,), jnp.int32),
            pltpu.VMEM((e // _NW,), jnp.int32),
            pltpu.VMEM((e // _NW,), jnp.int32),
            pltpu.VMEM((_C, _F), jnp.float32),
            pltpu.VMEM((rf.shape[0],), jnp.float32),
            pltpu.VMEM((e // _NW,), jnp.float32),
            pltpu.SemaphoreType.DMA,
        ],
        compiler_params=pltpu.CompilerParams(needs_layout_passes=False),
    )
    return fn(emb, zp, rf, idx_i, idx_j)


def _sc_msg_body(hp_hbm, xj_hbm, ii3_hbm, jj3_hbm, zeros_hbm, out_hbm,
                 ia, ja, ib, jb, hva, xva, hvb, xvb, msg_sp,
                 sem_ha, sem_xa, sem_hb, sem_xb):
    c = lax.axis_index("c")
    s = lax.axis_index("s")
    wid = c * 16 + s

    n_acc = zeros_hbm.shape[0]
    rows_per_sub = n_acc // 16
    ch = ii3_hbm.shape[1]
    base = wid * ch * _C

    pltpu.sync_copy(zeros_hbm.at[pl.ds(s * rows_per_sub, rows_per_sub)],
                    msg_sp.at[pl.ds(s * rows_per_sub, rows_per_sub)])
    plsc.subcore_barrier()

    pltpu.sync_copy(ii3_hbm.at[wid, 0], ia)
    pltpu.sync_copy(jj3_hbm.at[wid, 0], ja)
    pltpu.async_copy(xj_hbm.at[ja], xva, sem_xa)
    pltpu.async_copy(hp_hbm.at[pl.ds(base, _C)], hva, sem_ha)
    pltpu.sync_copy(ii3_hbm.at[wid, 1], ib)
    pltpu.sync_copy(jj3_hbm.at[wid, 1], jb)
    pltpu.async_copy(xj_hbm.at[jb], xvb, sem_xb)
    pltpu.async_copy(hp_hbm.at[pl.ds(base + _C, _C)], hvb, sem_hb)

    def halfstep(k, iv, jv, hv, xv, sem_h, sem_x):
        pltpu.make_async_copy(hp_hbm.at[pl.ds(base, _C)], hv, sem_h).wait()
        pltpu.make_async_copy(xj_hbm.at[pl.ds(0, _C)], xv, sem_x).wait()

        def mul_row(r, carry2):
            for g in range(_F // 32):
                vi = hv[r, pl.ds(g * 16, 16)]
                lo = plsc.bitcast(vi << 16, jnp.float32)
                hi = plsc.bitcast(vi & jnp.int32(-65536), jnp.float32)
                sl_lo = pl.ds(g * 16, 16)
                sl_hi = pl.ds(64 + g * 16, 16)
                xv[r, sl_lo] = lo * xv[r, sl_lo]
                xv[r, sl_hi] = hi * xv[r, sl_hi]
            return carry2

        lax.fori_loop(0, _C, mul_row, 0)
        pltpu.sync_copy(xv, msg_sp.at[iv], add=True)

        @pl.when(k + 2 < ch)
        def _():
            pltpu.sync_copy(ii3_hbm.at[wid, k + 2], iv)
            pltpu.sync_copy(jj3_hbm.at[wid, k + 2], jv)
            pltpu.async_copy(xj_hbm.at[jv], xv, sem_x)
            pltpu.async_copy(hp_hbm.at[pl.ds(base + (k + 2) * _C, _C)],
                             hv, sem_h)

    def body(m, carry):
        halfstep(2 * m, ia, ja, hva, xva, sem_ha, sem_xa)
        halfstep(2 * m + 1, ib, jb, hvb, xvb, sem_hb, sem_xb)
        return carry

    lax.fori_loop(0, ch // 2, body, 0)
    halfstep(ch - 1, ia, ja, hva, xva, sem_ha, sem_xa)
    plsc.subcore_barrier()
    pltpu.sync_copy(msg_sp.at[pl.ds(s * rows_per_sub, rows_per_sub)],
                    out_hbm.at[c, pl.ds(s * rows_per_sub, rows_per_sub)])


def _sc_msg(hp, xj, ii3, jj3, zeros):
    n_acc = zeros.shape[0]
    mesh = plsc.VectorSubcoreMesh(core_axis_name="c", subcore_axis_name="s")
    fn = pl.kernel(
        _sc_msg_body,
        out_type=jax.ShapeDtypeStruct((2, n_acc, _F), jnp.float32),
        mesh=mesh,
        scratch_types=[
            pltpu.VMEM((_C,), jnp.int32),
            pltpu.VMEM((_C,), jnp.int32),
            pltpu.VMEM((_C,), jnp.int32),
            pltpu.VMEM((_C,), jnp.int32),
            pltpu.VMEM((_C, _F // 2), jnp.int32),
            pltpu.VMEM((_C, _F), jnp.float32),
            pltpu.VMEM((_C, _F // 2), jnp.int32),
            pltpu.VMEM((_C, _F), jnp.float32),
            pltpu.VMEM_SHARED((n_acc, _F), jnp.float32),
            pltpu.SemaphoreType.DMA,
            pltpu.SemaphoreType.DMA,
            pltpu.SemaphoreType.DMA,
            pltpu.SemaphoreType.DMA,
        ],
        compiler_params=pltpu.CompilerParams(needs_layout_passes=False),
    )
    return fn(hp, xj, ii3, jj3, zeros)


def _tc_pre_body(x_ref, w_ref, b_ref, xj_ref):
    xa = _ssp(x_ref[...])
    xj_ref[...] = _ssp(jnp.dot(xa, w_ref[...],
                               preferred_element_type=jnp.float32) + b_ref[...])


def _tc_pre(x, w, b):
    n = x.shape[0]
    grid = (n // _N_TILE,)
    return pl.pallas_call(
        _tc_pre_body,
        grid=grid,
        in_specs=[
            pl.BlockSpec((_N_TILE, _F), lambda i: (i, 0)),
            pl.BlockSpec((_F, _F), lambda i: (0, 0)),
            pl.BlockSpec((1, _F), lambda i: (0, 0)),
        ],
        out_specs=pl.BlockSpec((_N_TILE, _F), lambda i: (i, 0)),
        out_shape=jax.ShapeDtypeStruct((n, _F), jnp.float32),
    )(x, w, b)


def _tc_h_body(d2_ref, cen_ref, wid_ref, k2f_ref, h_ref):
    d2 = jnp.reshape(d2_ref[...], (1, _E_TILE))
    d = jnp.sqrt(d2 + 1e-12)
    u = d * (1.0 / _CUTOFF)
    u3 = u * u * u
    f = 1.0 - 6.0 * u3 * u * u + 15.0 * u3 * u - 10.0 * u3
    cf = jnp.where(d < _CUTOFF, f, 0.0)
    t = jnp.exp(-d) - cen_ref[...]
    rbf_t = cf * jnp.exp(-wid_ref[...] * t * t)
    h = lax.dot_general(rbf_t, k2f_ref[...], (((0,), (0,)), ((), ())),
                        preferred_element_type=jnp.float32)
    u = lax.bitcast_convert_type(h, jnp.int32)
    t = lax.shift_right_logical(
        u + 0x7FFF + (lax.shift_right_logical(u, 16) & 1), 16)
    h_ref[...] = t[:, : _F // 2] | lax.shift_left(t[:, _F // 2:], 16)


def _tc_h(d2, cen, wid, k2f):
    e = d2.shape[0]
    grid = (e // _E_TILE,)
    return pl.pallas_call(
        _tc_h_body,
        grid=grid,
        in_specs=[
            pl.BlockSpec((_E_TILE,), lambda i: (i,)),
            pl.BlockSpec((_K, 1), lambda i: (0, 0)),
            pl.BlockSpec((_K, 1), lambda i: (0, 0)),
            pl.BlockSpec((_K, _F), lambda i: (0, 0)),
        ],
        out_specs=pl.BlockSpec((_E_TILE, _F // 2), lambda i: (i, 0)),
        out_shape=jax.ShapeDtypeStruct((e, _F // 2), jnp.int32),
    )(d2, cen, wid, k2f)


def _tc_post_body(x_ref, m_ref, *refs):
    wrefs = refs[:-2]
    xn_ref, out_ref = refs[-2:]
    ws = [r[...] for r in wrefs]
    (wdi, bdi,
     ri0w1, ri0b1, ri0w2, ri0b2, ri1w1, ri1b1, ri1w2, ri1b2,
     wd, bd, u,
     ra0w1, ra0b1, ra0w2, ra0b2, ra1w1, ra1b1, ra1w2, ra1b2,
     ro0w1, ro0b1, ro0w2, ro0b2,
     wout, bout) = ws

    x = x_ref[...]
    xa = _ssp(x)
    xi = _ssp(jnp.dot(xa, wdi, preferred_element_type=jnp.float32) + bdi)
    m = xi + m_ref[0] + m_ref[1]
    m = _residual(m, ri0w1, ri0b1, ri0w2, ri0b2)
    m = _residual(m, ri1w1, ri1b1, ri1w2, ri1b2)
    m = _ssp(m)
    xn = u * x + jnp.dot(m, wd, preferred_element_type=jnp.float32) + bd
    xn = _residual(xn, ra0w1, ra0b1, ra0w2, ra0b2)
    xn = _residual(xn, ra1w1, ra1b1, ra1w2, ra1b2)
    xn_ref[...] = xn
    t = _residual(xn, ro0w1, ro0b1, ro0w2, ro0b2)
    t = _ssp(t)
    out_ref[...] = jnp.dot(t, wout, preferred_element_type=jnp.float32) + bout


def _tc_post(x, msgp, wlist):
    n = x.shape[0]
    grid = (n // _N_TILE,)
    wspecs = [pl.BlockSpec(w.shape, lambda i: (0,) * w.ndim) for w in wlist]
    return pl.pallas_call(
        _tc_post_body,
        grid=grid,
        in_specs=[
            pl.BlockSpec((_N_TILE, _F), lambda i: (i, 0)),
            pl.BlockSpec((2, _N_TILE, _F), lambda i: (0, i, 0)),
        ] + wspecs,
        out_specs=[
            pl.BlockSpec((_N_TILE, _F), lambda i: (i, 0)),
            pl.BlockSpec((_N_TILE, 8), lambda i: (i, 0)),
        ],
        out_shape=[
            jax.ShapeDtypeStruct((n, _F), jnp.float32),
            jax.ShapeDtypeStruct((n, 8), jnp.float32),
        ],
    )(x, msgp, *wlist)


def _tc_final_body(o1_ref, o2_ref, o3_ref, eq_ref, nh_ref):
    o1 = o1_ref[...]
    o2 = o2_ref[...]
    o3 = o3_ref[...]
    eq_ref[...] = o1 + o2 + o3
    s1 = o1 * o1
    s2 = o2 * o2
    s3 = o3 * o3
    n2 = o1.shape[0] * 2.0
    r2 = s2 / (s2 + s1 + 1e-7)
    r3 = s3 / (s3 + s2 + 1e-7)
    nh_ref[...] = jnp.reshape((jnp.sum(r2) + jnp.sum(r3)) / n2, (1, 1))


def _tc_final(o1, o2, o3):
    n = o1.shape[0]
    return pl.pallas_call(
        _tc_final_body,
        out_shape=[
            jax.ShapeDtypeStruct((n, 8), jnp.float32),
            jax.ShapeDtypeStruct((1, 1), jnp.float32),
        ],
    )(o1, o2, o3)


def _post_weights(blk):
    ws = [blk["di"]["W"], blk["di"]["b"].reshape(1, _F)]
    for r in blk["res_int"]:
        ws += [r["d1"]["W"], r["d1"]["b"].reshape(1, _F),
               r["d2"]["W"], r["d2"]["b"].reshape(1, _F)]
    ws += [blk["dense"]["W"], blk["dense"]["b"].reshape(1, _F),
           blk["u"].reshape(1, _F)]
    for r in blk["res_atomic"]:
        ws += [r["d1"]["W"], r["d1"]["b"].reshape(1, _F),
               r["d2"]["W"], r["d2"]["b"].reshape(1, _F)]
    for r in blk["res_out"]:
        ws += [r["d1"]["W"], r["d1"]["b"].reshape(1, _F),
               r["d2"]["W"], r["d2"]["b"].reshape(1, _F)]
    wout = jnp.pad(blk["out"]["W"], ((0, 0), (0, 8 - blk["out"]["W"].shape[1])))
    bout = jnp.pad(blk["out"]["b"], (0, 8 - blk["out"]["b"].shape[0]))
    ws += [wout, bout.reshape(1, 8)]
    return ws


def kernel(Z, R, idx_i, idx_j, params):
    n = Z.shape[0]

    n_pad = ((n + _C * _NW - 1) // (_C * _NW)) * (_C * _NW)
    zp = jnp.pad(Z.astype(jnp.int32), (0, n_pad - n))
    rf = jnp.pad(R.astype(jnp.float32), ((0, 0), (0, 4 - R.shape[1]))).reshape(-1)
    idx_i = idx_i.astype(jnp.int32)
    idx_j = idx_j.astype(jnp.int32)
    n_acc = ((n + 127) // 128) * 128
    zeros = jnp.zeros((n_acc, _F), jnp.float32)
    e = idx_i.shape[0]
    ii3 = idx_i.reshape(_NW, e // (_NW * _C), _C)
    jj3 = idx_j.reshape(_NW, e // (_NW * _C), _C)
    cen = params["rbf_centers"].reshape(_K, 1)
    wid = params["rbf_widths"].reshape(_K, 1)

    x0p, d2 = _sc_setup(params["emb"], zp, rf, idx_i, idx_j)
    x = x0p[:n]
    e = idx_i.shape[0]
    ep = ((e + _E_TILE - 1) // _E_TILE) * _E_TILE
    d2p = jnp.pad(d2, (0, ep - e))

    outs = []
    for blk in params["blocks"]:
        xj = _tc_pre(x, blk["dj"]["W"], blk["dj"]["b"].reshape(1, _F))
        hp = _tc_h(d2p, cen, wid, blk["k2f"])
        msgp = _sc_msg(hp, xj, ii3, jj3, zeros)
        x, ob = _tc_post(x, msgp, _post_weights(blk))
        outs.append(ob)

    eq, nh = _tc_final(*outs)
    return eq[:, 0], eq[:, 1], nh[0, 0]
